# baseline (speedup 1.0000x reference)
import functools

import jax
import jax.numpy as jnp
from jax import lax
from jax.experimental import pallas as pl
from jax.experimental.pallas import tpu as pltpu
from jax.experimental.pallas import tpu_sc as plsc

_BATCH = 64
_NUM_PATCHES = 1024
_PROJ_DIM = 768
_NC = 2
_NS = 16
_NW = _NC * _NS
_ROWS_PER_W = _NUM_PATCHES // _NW
_CHUNK = _ROWS_PER_W * _PROJ_DIM
_VECS = _CHUNK // 16
_UNROLL = 8


def _sc_body(patches_hbm, pos_hbm, out_hbm, pos_v, buf_v, sem):
    wid = lax.axis_index("s") * _NC + lax.axis_index("c")
    off = wid * _CHUNK
    pltpu.sync_copy(pos_hbm.at[pl.ds(off, _CHUNK)], pos_v)

    def batch_body(b, carry):
        pltpu.sync_copy(patches_hbm.at[b, pl.ds(off, _CHUNK)], buf_v)

        def add_body(i, c2):
            base = i * (16 * _UNROLL)
            for j in range(_UNROLL):
                s = base + j * 16
                buf_v[pl.ds(s, 16)] = buf_v[pl.ds(s, 16)] + pos_v[pl.ds(s, 16)]
            return c2

        lax.fori_loop(0, _VECS // _UNROLL, add_body, 0)
        pltpu.sync_copy(buf_v, out_hbm.at[b, pl.ds(off, _CHUNK)])
        return carry

    lax.fori_loop(0, _BATCH, batch_body, 0)


def kernel(patches, pos_table):
    batch, num_patches, proj_dim = patches.shape
    patches2d = patches.reshape(batch, num_patches * proj_dim)
    pos_flat = pos_table.reshape(num_patches * proj_dim)

    mesh = plsc.VectorSubcoreMesh(core_axis_name="c", subcore_axis_name="s")
    run = functools.partial(
        pl.kernel,
        out_type=jax.ShapeDtypeStruct((batch, num_patches * proj_dim), jnp.float32),
        mesh=mesh,
        scratch_types=[
            pltpu.VMEM((_CHUNK,), jnp.float32),
            pltpu.VMEM((_CHUNK,), jnp.float32),
            pltpu.SemaphoreType.DMA,
        ],
    )(_sc_body)
    out2d = run(patches2d, pos_flat)
    return out2d.reshape(batch, num_patches, proj_dim)
